# baseline (speedup 1.0000x reference)
import jax
import jax.numpy as jnp
from jax.experimental import pallas as pl
from jax.experimental.pallas import tpu as pltpu

_KP = 128
_C = 4


def _body(f_ref, sc_ref, phc_ref, out_ref,
          fh_ref, hn_ref, sum_ref, hi_ref, cn_ref,
          div_ref, nu_ref, ts_ref, tc_ref):
    ph = pl.program_id(0)
    c = pl.program_id(1)
    nc, d = f_ref.shape
    sc = sc_ref[...]
    phc = phc_ref[...]
    lane = jax.lax.broadcasted_iota(jnp.int32, (nc, _KP), 1)
    E = (lane == phc).astype(jnp.float32)
    ones_n = jnp.ones((nc, 1), jnp.float32)

    @pl.when(jnp.logical_and(ph == 0, c == 0))
    def _init():
        sum_ref[...] = jnp.zeros_like(sum_ref)
        hi_ref[...] = jnp.zeros_like(hi_ref)
        cn_ref[...] = jnp.zeros_like(cn_ref)

    @pl.when(ph == 0)
    def _phase0():
        F = f_ref[...]
        m2 = jnp.where(sc == 2.0, 1.0, 0.0)
        sum_ref[...] += jax.lax.dot_general(
            E, F, (((0,), (0,)), ((), ())),
            preferred_element_type=jnp.float32)
        hi_ref[...] += jax.lax.dot_general(
            E, m2, (((0,), (0,)), ((), ())),
            preferred_element_type=jnp.float32)
        cn_ref[...] += jax.lax.dot_general(
            E, ones_n, (((0,), (0,)), ((), ())),
            preferred_element_type=jnp.float32)
        ones_d = jnp.ones((d, 1), jnp.float32)
        fn2 = jax.lax.dot_general(
            F * F, ones_d, (((1,), (0,)), ((), ())),
            preferred_element_type=jnp.float32)
        rs2 = 1.0 / jnp.maximum(fn2, 1e-24)
        rs = jnp.sqrt(rs2)
        fh_ref[pl.ds(c * nc, nc), :] = F * rs
        hn_ref[pl.ds(c * nc, nc), :] = fn2 * rs2

    @pl.when(jnp.logical_and(ph == 1, c == 0))
    def _boundary():
        hp = jnp.where(hi_ref[...] > 0.0, 1.0, 0.0)
        counts = cn_ref[...] * hp
        center = (sum_ref[...] * hp) / jnp.maximum(counts, 1.0)
        nrm = jnp.sqrt(jnp.sum(center * center, axis=1, keepdims=True))
        center = center / jnp.maximum(nrm, 1e-12)
        nrm2 = jnp.sqrt(jnp.sum(center * center, axis=1, keepdims=True))
        p = center / jnp.maximum(nrm2, 1e-12)
        pn2 = jnp.sum(p * p, axis=1, keepdims=True)
        Gpp = jax.lax.dot_general(
            p, p, (((1,), (1,)), ((), ())),
            preferred_element_type=jnp.float32)
        ii = jax.lax.broadcasted_iota(jnp.int32, (_KP, _KP), 0)
        jj = jax.lax.broadcasted_iota(jnp.int32, (_KP, _KP), 1)
        d2 = pn2 + jnp.transpose(pn2) - 2.0 * Gpp
        dist = jnp.sqrt(jnp.maximum(d2, 1e-12))
        pairp = jax.lax.dot_general(
            hp, hp, (((1,), (1,)), ((), ())),
            preferred_element_type=jnp.float32)
        pair = (pairp > 0.5) & (ii < jj)
        n_u = jnp.sum(hp)
        denom = jnp.maximum(n_u * (n_u - 1.0) * 0.5, 1.0)
        diversity = jnp.sum(jnp.where(pair, dist, 0.0)) / denom
        div_ref[...] = jnp.broadcast_to(diversity, (1, 1))
        nu_ref[...] = jnp.broadcast_to(n_u, (1, 1))
        ts_ref[...] = jnp.zeros_like(ts_ref)
        tc_ref[...] = jnp.zeros_like(tc_ref)
        sum_ref[...] = p
        hi_ref[...] = hp

    @pl.when(ph == 1)
    def _phase1():
        fhat = fh_ref[pl.ds(c * nc, nc), :]
        hn2 = hn_ref[pl.ds(c * nc, nc), :]
        keepc = jax.lax.dot_general(
            E, hi_ref[...], (((1,), (0,)), ((), ())),
            preferred_element_type=jnp.float32)
        p_sel = jax.lax.dot_general(
            E, sum_ref[...], (((1,), (0,)), ((), ())),
            preferred_element_type=jnp.float32)
        ones_d = jnp.ones((d, 1), jnp.float32)
        dotF = jax.lax.dot_general(
            fhat * p_sel, ones_d, (((1,), (0,)), ((), ())),
            preferred_element_type=jnp.float32)
        dsq = hn2 + keepc - 2.0 * dotF
        nzf = jnp.where(dsq > 0.0, keepc, 0.0)
        tw = jnp.sqrt(jnp.maximum(dsq, 0.0)) * (3.0 - sc)
        ts_ref[...] += jax.lax.dot_general(
            tw, nzf, (((0,), (0,)), ((), ())),
            preferred_element_type=jnp.float32)
        tc_ref[...] += jax.lax.dot_general(
            nzf, ones_n, (((0,), (0,)), ((), ())),
            preferred_element_type=jnp.float32)

    @pl.when(jnp.logical_and(ph == 1, c == pl.num_programs(1) - 1))
    def _final():
        tight = ts_ref[...] / jnp.maximum(tc_ref[...], 1.0)
        loss = 0.1 * tight - 0.5 * div_ref[...]
        out_ref[...] = jnp.where(nu_ref[...] >= 2.0, loss, 0.0)


def kernel(features, scores, phn_ids):
    B, T, D = features.shape
    N = B * T
    nc = N // _C
    F = features.reshape(N, D)
    sc = scores.reshape(N, 1)
    phc = phn_ids.reshape(N, 1).astype(jnp.int32)
    out = pl.pallas_call(
        _body,
        grid=(2, _C),
        in_specs=[
            pl.BlockSpec((nc, D), lambda ph, c: (jnp.where(ph == 0, c, _C - 1), 0)),
            pl.BlockSpec((nc, 1), lambda ph, c: (c, 0)),
            pl.BlockSpec((nc, 1), lambda ph, c: (c, 0)),
        ],
        out_specs=pl.BlockSpec((1, 1), lambda ph, c: (0, 0)),
        out_shape=jax.ShapeDtypeStruct((1, 1), jnp.float32),
        scratch_shapes=[
            pltpu.VMEM((N, D), jnp.float32),
            pltpu.VMEM((N, 1), jnp.float32),
            pltpu.VMEM((_KP, D), jnp.float32),
            pltpu.VMEM((_KP, 1), jnp.float32),
            pltpu.VMEM((_KP, 1), jnp.float32),
            pltpu.VMEM((1, 1), jnp.float32),
            pltpu.VMEM((1, 1), jnp.float32),
            pltpu.VMEM((1, 1), jnp.float32),
            pltpu.VMEM((1, 1), jnp.float32),
        ],
        compiler_params=pltpu.CompilerParams(
            dimension_semantics=("arbitrary", "arbitrary")),
    )(F, sc, phc)
    return out[0, 0]


# speedup vs baseline: 1.0531x; 1.0531x over previous
import jax
import jax.numpy as jnp
from jax.experimental import pallas as pl
from jax.experimental.pallas import tpu as pltpu

_KP = 128
_C = 2


def _body(f_ref, sc_ref, phc_ref, out_ref,
          fh_ref, hn_ref, e_ref, w_ref, sum_ref, hi_ref, cn_ref,
          div_ref, nu_ref, ts_ref, tc_ref):
    ph = pl.program_id(0)
    c = pl.program_id(1)
    nc, d = f_ref.shape
    sc = sc_ref[...]
    phc = phc_ref[...]
    ones_n = jnp.ones((nc, 1), jnp.float32)

    @pl.when(jnp.logical_and(ph == 0, c == 0))
    def _init():
        sum_ref[...] = jnp.zeros_like(sum_ref)
        hi_ref[...] = jnp.zeros_like(hi_ref)
        cn_ref[...] = jnp.zeros_like(cn_ref)

    @pl.when(ph == 0)
    def _phase0():
        F = f_ref[...]
        lane = jax.lax.broadcasted_iota(jnp.int32, (nc, _KP), 1)
        E = (lane == phc).astype(jnp.float32)
        e_ref[pl.ds(c * nc, nc), :] = E
        w_ref[pl.ds(c * nc, nc), :] = 3.0 - sc
        m2 = jnp.where(sc == 2.0, 1.0, 0.0)
        sum_ref[...] += jax.lax.dot_general(
            E, F, (((0,), (0,)), ((), ())),
            preferred_element_type=jnp.float32)
        hi_ref[...] += jax.lax.dot_general(
            E, m2, (((0,), (0,)), ((), ())),
            preferred_element_type=jnp.float32)
        cn_ref[...] += jax.lax.dot_general(
            E, ones_n, (((0,), (0,)), ((), ())),
            preferred_element_type=jnp.float32)
        ones_d = jnp.ones((d, 1), jnp.float32)
        fn2 = jax.lax.dot_general(
            F * F, ones_d, (((1,), (0,)), ((), ())),
            preferred_element_type=jnp.float32)
        rs2 = 1.0 / jnp.maximum(fn2, 1e-24)
        rs = jnp.sqrt(rs2)
        fh_ref[pl.ds(c * nc, nc), :] = F * rs
        hn_ref[pl.ds(c * nc, nc), :] = fn2 * rs2

    @pl.when(jnp.logical_and(ph == 1, c == 0))
    def _boundary():
        hp = jnp.where(hi_ref[...] > 0.0, 1.0, 0.0)
        counts = cn_ref[...] * hp
        center = (sum_ref[...] * hp) / jnp.maximum(counts, 1.0)
        nrm = jnp.sqrt(jnp.sum(center * center, axis=1, keepdims=True))
        center = center / jnp.maximum(nrm, 1e-12)
        nrm2 = jnp.sqrt(jnp.sum(center * center, axis=1, keepdims=True))
        p = center / jnp.maximum(nrm2, 1e-12)
        pn2 = jnp.sum(p * p, axis=1, keepdims=True)
        Gpp = jax.lax.dot_general(
            p, p, (((1,), (1,)), ((), ())),
            preferred_element_type=jnp.float32)
        ii = jax.lax.broadcasted_iota(jnp.int32, (_KP, _KP), 0)
        jj = jax.lax.broadcasted_iota(jnp.int32, (_KP, _KP), 1)
        d2 = pn2 + jnp.transpose(pn2) - 2.0 * Gpp
        dist = jnp.sqrt(jnp.maximum(d2, 1e-12))
        pairp = jax.lax.dot_general(
            hp, hp, (((1,), (1,)), ((), ())),
            preferred_element_type=jnp.float32)
        pair = (pairp > 0.5) & (ii < jj)
        n_u = jnp.sum(hp)
        denom = jnp.maximum(n_u * (n_u - 1.0) * 0.5, 1.0)
        diversity = jnp.sum(jnp.where(pair, dist, 0.0)) / denom
        div_ref[...] = jnp.broadcast_to(diversity, (1, 1))
        nu_ref[...] = jnp.broadcast_to(n_u, (1, 1))
        ts_ref[...] = jnp.zeros_like(ts_ref)
        tc_ref[...] = jnp.zeros_like(tc_ref)
        sum_ref[...] = p
        hi_ref[...] = hp

    @pl.when(ph == 1)
    def _phase1():
        fhat = fh_ref[pl.ds(c * nc, nc), :]
        hn2 = hn_ref[pl.ds(c * nc, nc), :]
        E = e_ref[pl.ds(c * nc, nc), :]
        keepc = jax.lax.dot_general(
            E, hi_ref[...], (((1,), (0,)), ((), ())),
            preferred_element_type=jnp.float32)
        p_sel = jax.lax.dot_general(
            E, sum_ref[...], (((1,), (0,)), ((), ())),
            preferred_element_type=jnp.float32)
        ones_d = jnp.ones((d, 1), jnp.float32)
        dotF = jax.lax.dot_general(
            fhat * p_sel, ones_d, (((1,), (0,)), ((), ())),
            preferred_element_type=jnp.float32)
        dsq = hn2 + keepc - 2.0 * dotF
        nzf = jnp.where(dsq > 0.0, keepc, 0.0)
        tw = jnp.sqrt(jnp.maximum(dsq, 0.0)) * w_ref[pl.ds(c * nc, nc), :]
        ts_ref[...] += jax.lax.dot_general(
            tw, nzf, (((0,), (0,)), ((), ())),
            preferred_element_type=jnp.float32)
        tc_ref[...] += jax.lax.dot_general(
            nzf, ones_n, (((0,), (0,)), ((), ())),
            preferred_element_type=jnp.float32)

    @pl.when(jnp.logical_and(ph == 1, c == pl.num_programs(1) - 1))
    def _final():
        tight = ts_ref[...] / jnp.maximum(tc_ref[...], 1.0)
        loss = 0.1 * tight - 0.5 * div_ref[...]
        out_ref[...] = jnp.where(nu_ref[...] >= 2.0, loss, 0.0)


def kernel(features, scores, phn_ids):
    B, T, D = features.shape
    N = B * T
    nc = N // _C
    F = features.reshape(N, D)
    sc = scores.reshape(N, 1)
    phc = phn_ids.reshape(N, 1).astype(jnp.int32)
    out = pl.pallas_call(
        _body,
        grid=(2, _C),
        in_specs=[
            pl.BlockSpec((nc, D), lambda ph, c: (jnp.where(ph == 0, c, _C - 1), 0)),
            pl.BlockSpec((nc, 1), lambda ph, c: (c, 0)),
            pl.BlockSpec((nc, 1), lambda ph, c: (c, 0)),
        ],
        out_specs=pl.BlockSpec((1, 1), lambda ph, c: (0, 0)),
        out_shape=jax.ShapeDtypeStruct((1, 1), jnp.float32),
        scratch_shapes=[
            pltpu.VMEM((N, D), jnp.float32),
            pltpu.VMEM((N, 1), jnp.float32),
            pltpu.VMEM((N, _KP), jnp.float32),
            pltpu.VMEM((N, 1), jnp.float32),
            pltpu.VMEM((_KP, D), jnp.float32),
            pltpu.VMEM((_KP, 1), jnp.float32),
            pltpu.VMEM((_KP, 1), jnp.float32),
            pltpu.VMEM((1, 1), jnp.float32),
            pltpu.VMEM((1, 1), jnp.float32),
            pltpu.VMEM((1, 1), jnp.float32),
            pltpu.VMEM((1, 1), jnp.float32),
        ],
        compiler_params=pltpu.CompilerParams(
            dimension_semantics=("arbitrary", "arbitrary")),
    )(F, sc, phc)
    return out[0, 0]


# speedup vs baseline: 1.1574x; 1.0991x over previous
import jax
import jax.numpy as jnp
from jax.experimental import pallas as pl
from jax.experimental.pallas import tpu as pltpu

_KP = 128
_C = 2


def _body(f_ref, sc_ref, phc_ref, out_ref,
          fh_ref, hn_ref, e_ref, w_ref, sum_ref, hi_ref, cn_ref):
    g = pl.program_id(0)
    nc, d = f_ref.shape
    ones_d = jnp.ones((d, 1), jnp.float32)

    @pl.when(g == 0)
    def _init():
        sum_ref[...] = jnp.zeros_like(sum_ref)
        hi_ref[...] = jnp.zeros_like(hi_ref)
        cn_ref[...] = jnp.zeros_like(cn_ref)

    @pl.when(g < _C)
    def _stream():
        F = f_ref[...]
        sc = sc_ref[...]
        phc = phc_ref[...]
        lane = jax.lax.broadcasted_iota(jnp.int32, (nc, _KP), 1)
        E = (lane == phc).astype(jnp.float32)
        e_ref[pl.ds(g * nc, nc), :] = E
        w_ref[pl.ds(g * nc, nc), :] = 3.0 - sc
        m2 = jnp.where(sc == 2.0, 1.0, 0.0)
        ones_n = jnp.ones((nc, 1), jnp.float32)
        sum_ref[...] += jax.lax.dot_general(
            E, F, (((0,), (0,)), ((), ())),
            preferred_element_type=jnp.float32)
        hi_ref[...] += jax.lax.dot_general(
            E, m2, (((0,), (0,)), ((), ())),
            preferred_element_type=jnp.float32)
        cn_ref[...] += jax.lax.dot_general(
            E, ones_n, (((0,), (0,)), ((), ())),
            preferred_element_type=jnp.float32)
        fn2 = jax.lax.dot_general(
            F * F, ones_d, (((1,), (0,)), ((), ())),
            preferred_element_type=jnp.float32)
        rs2 = 1.0 / jnp.maximum(fn2, 1e-24)
        rs = jnp.sqrt(rs2)
        fh_ref[pl.ds(g * nc, nc), :] = F * rs
        hn_ref[pl.ds(g * nc, nc), :] = fn2 * rs2

    @pl.when(g == _C)
    def _reduce():
        hp = jnp.where(hi_ref[...] > 0.0, 1.0, 0.0)
        counts = cn_ref[...] * hp
        center = (sum_ref[...] * hp) / jnp.maximum(counts, 1.0)
        nrm = jnp.sqrt(jnp.sum(center * center, axis=1, keepdims=True))
        center = center / jnp.maximum(nrm, 1e-12)
        nrm2 = jnp.sqrt(jnp.sum(center * center, axis=1, keepdims=True))
        p = center / jnp.maximum(nrm2, 1e-12)
        pn2 = jnp.sum(p * p, axis=1, keepdims=True)
        Gpp = jax.lax.dot_general(
            p, p, (((1,), (1,)), ((), ())),
            preferred_element_type=jnp.float32)
        ii = jax.lax.broadcasted_iota(jnp.int32, (_KP, _KP), 0)
        jj = jax.lax.broadcasted_iota(jnp.int32, (_KP, _KP), 1)
        d2 = pn2 + jnp.transpose(pn2) - 2.0 * Gpp
        dist = jnp.sqrt(jnp.maximum(d2, 1e-12))
        pairp = jax.lax.dot_general(
            hp, hp, (((1,), (1,)), ((), ())),
            preferred_element_type=jnp.float32)
        pair = (pairp > 0.5) & (ii < jj)
        n_u = jnp.sum(hp)
        denom = jnp.maximum(n_u * (n_u - 1.0) * 0.5, 1.0)
        diversity = jnp.sum(jnp.where(pair, dist, 0.0)) / denom

        fhat = fh_ref[...]
        hn2 = hn_ref[...]
        E = e_ref[...]
        n = fhat.shape[0]
        keepc = jax.lax.dot_general(
            E, hp, (((1,), (0,)), ((), ())),
            preferred_element_type=jnp.float32)
        p_sel = jax.lax.dot_general(
            E, p, (((1,), (0,)), ((), ())),
            preferred_element_type=jnp.float32)
        dotF = jax.lax.dot_general(
            fhat * p_sel, ones_d, (((1,), (0,)), ((), ())),
            preferred_element_type=jnp.float32)
        dsq = hn2 + keepc - 2.0 * dotF
        nzf = jnp.where(dsq > 0.0, keepc, 0.0)
        tw = jnp.sqrt(jnp.maximum(dsq, 0.0)) * w_ref[...]
        ts = jax.lax.dot_general(
            tw, nzf, (((0,), (0,)), ((), ())),
            preferred_element_type=jnp.float32)
        tc = jax.lax.dot_general(
            nzf, jnp.ones((n, 1), jnp.float32), (((0,), (0,)), ((), ())),
            preferred_element_type=jnp.float32)
        tight = ts / jnp.maximum(tc, 1.0)
        loss = 0.1 * tight - 0.5 * diversity
        out_ref[...] = jnp.where(n_u >= 2.0, loss, jnp.zeros((1, 1), jnp.float32))


def kernel(features, scores, phn_ids):
    B, T, D = features.shape
    N = B * T
    nc = N // _C
    F = features.reshape(N, D)
    sc = scores.reshape(N, 1)
    phc = phn_ids.reshape(N, 1).astype(jnp.int32)
    out = pl.pallas_call(
        _body,
        grid=(_C + 1,),
        in_specs=[
            pl.BlockSpec((nc, D), lambda g: (jnp.minimum(g, _C - 1), 0)),
            pl.BlockSpec((nc, 1), lambda g: (jnp.minimum(g, _C - 1), 0)),
            pl.BlockSpec((nc, 1), lambda g: (jnp.minimum(g, _C - 1), 0)),
        ],
        out_specs=pl.BlockSpec((1, 1), lambda g: (0, 0)),
        out_shape=jax.ShapeDtypeStruct((1, 1), jnp.float32),
        scratch_shapes=[
            pltpu.VMEM((N, D), jnp.float32),
            pltpu.VMEM((N, 1), jnp.float32),
            pltpu.VMEM((N, _KP), jnp.float32),
            pltpu.VMEM((N, 1), jnp.float32),
            pltpu.VMEM((_KP, D), jnp.float32),
            pltpu.VMEM((_KP, 1), jnp.float32),
            pltpu.VMEM((_KP, 1), jnp.float32),
        ],
        compiler_params=pltpu.CompilerParams(
            dimension_semantics=("arbitrary",)),
    )(F, sc, phc)
    return out[0, 0]
